# Initial kernel scaffold; baseline (speedup 1.0000x reference)
import jax
import jax.numpy as jnp
from jax.experimental import pallas as pl


def kernel(x, w1, b1, g1, be1, w2, b2, g2, be2):
    raise NotImplementedError("write your pallas kernel here")


# speedup vs baseline: 1.3911x; 1.3911x over previous
import functools

import jax
import jax.numpy as jnp
import numpy as np
from jax import lax
from jax.experimental import pallas as pl
from jax.experimental.pallas import tpu as pltpu

_BN_EPS = 1e-5
_INV_SQRT2 = np.float32(1.0 / np.sqrt(2.0))
_PAD = 128


def _gelu(y):
    return 0.5 * y * (1.0 + lax.erf(y * _INV_SQRT2))


def _taps_cmajor(xb, w):
    c, hw = xb.shape
    zpad = jnp.zeros((c, _PAD), xb.dtype)
    xp = jnp.concatenate([zpad, xb, zpad], axis=1)
    wi = lax.broadcasted_iota(jnp.int32, (c, hw), 1) % w
    lmask = wi >= 1
    rmask = wi <= (w - 2)
    zero = jnp.zeros((), xb.dtype)
    taps = []
    for dy in range(3):
        for dx in range(3):
            o = (dy - 1) * w + (dx - 1)
            t = xp[:, _PAD + o:_PAD + o + hw]
            if dx == 0:
                t = jnp.where(lmask, t, zero)
            elif dx == 2:
                t = jnp.where(rmask, t, zero)
            taps.append(t)
    return taps


def _taps_rmajor(yb, w):
    hw, c = yb.shape
    zpad = jnp.zeros((_PAD, c), yb.dtype)
    yp = jnp.concatenate([zpad, yb, zpad], axis=0)
    wi = lax.broadcasted_iota(jnp.int32, (hw, c), 0) % w
    lmask = wi >= 1
    rmask = wi <= (w - 2)
    zero = jnp.zeros((), yb.dtype)
    taps = []
    for dy in range(3):
        for dx in range(3):
            o = (dy - 1) * w + (dx - 1)
            t = yp[_PAD + o:_PAD + o + hw, :]
            if dx == 0:
                t = jnp.where(lmask, t, zero)
            elif dx == 2:
                t = jnp.where(rmask, t, zero)
            taps.append(t)
    return taps


def _stats_rows(acc):
    s = jnp.sum(acc, axis=0, keepdims=True)
    ss = jnp.sum(acc * acc, axis=0, keepdims=True)
    return jnp.concatenate(
        [s, ss, jnp.zeros((6, acc.shape[1]), jnp.float32)], axis=0)


def _scale_shift(st_all, g, b, m):
    st = jnp.sum(st_all, axis=0)
    mean = st[0:1, :] / m
    var = jnp.maximum(st[1:2, :] / m - mean * mean, 0.0)
    inv = lax.rsqrt(var + _BN_EPS)
    scale = g * inv
    shift = b - mean * scale
    return scale, shift


def _conv1_kernel(x_ref, w_ref, o_ref, s_ref, *, w):
    xb = x_ref[0].astype(jnp.bfloat16)
    p = jnp.concatenate(_taps_cmajor(xb, w), axis=0)
    acc = lax.dot_general(p, w_ref[...], (((0,), (0,)), ((), ())),
                          preferred_element_type=jnp.float32)
    o_ref[0] = acc.astype(jnp.bfloat16)
    s_ref[0] = _stats_rows(acc)


def _conv2_kernel(h_ref, st_ref, g_ref, b_ref, w_ref, o_ref, s_ref, *, w, m):
    scale, shift = _scale_shift(st_ref[...], g_ref[...], b_ref[...], m)
    y = h_ref[0].astype(jnp.float32) * scale + shift
    yb = _gelu(y).astype(jnp.bfloat16)
    p = jnp.concatenate(_taps_rmajor(yb, w), axis=1)
    acc = jnp.dot(p, w_ref[...], preferred_element_type=jnp.float32)
    o_ref[0] = acc.astype(jnp.bfloat16)
    s_ref[0] = _stats_rows(acc)


def _pool_kernel(h_ref, st_ref, g_ref, b_ref, o_ref, *, h, w, m):
    scale, shift = _scale_shift(st_ref[...], g_ref[...], b_ref[...], m)
    y = _gelu(h_ref[0].astype(jnp.float32) * scale + shift)
    c = y.shape[1]
    y = jnp.max(y.reshape(h * (w // 2), 2, c), axis=1)
    y = jnp.max(y.reshape(h // 2, 2, (w // 2) * c), axis=1)
    o_ref[0] = y.reshape((h // 2) * (w // 2), c)


def kernel(x, w1, b1, g1, be1, w2, b2, g2, be2):
    n, cin, h, w = x.shape
    cout = g1.shape[0]
    hw = h * w
    m = float(n * hw)

    x2 = x.reshape(n, cin, hw)
    w1m = jnp.transpose(w1, (2, 3, 1, 0)).reshape(9 * cin, cout).astype(jnp.bfloat16)
    w2m = jnp.transpose(w2, (2, 3, 1, 0)).reshape(9 * cout, cout).astype(jnp.bfloat16)
    g1c, be1c = g1.reshape(1, cout), be1.reshape(1, cout)
    g2c, be2c = g2.reshape(1, cout), be2.reshape(1, cout)

    cparams = pltpu.CompilerParams(
        dimension_semantics=("parallel",),
        vmem_limit_bytes=100 * 1024 * 1024,
    )

    conv1, st1 = pl.pallas_call(
        functools.partial(_conv1_kernel, w=w),
        grid=(n,),
        in_specs=[pl.BlockSpec((1, cin, hw), lambda i: (i, 0, 0)),
                  pl.BlockSpec((9 * cin, cout), lambda i: (0, 0))],
        out_specs=[pl.BlockSpec((1, hw, cout), lambda i: (i, 0, 0)),
                   pl.BlockSpec((1, 8, cout), lambda i: (i, 0, 0))],
        out_shape=[jax.ShapeDtypeStruct((n, hw, cout), jnp.bfloat16),
                   jax.ShapeDtypeStruct((n, 8, cout), jnp.float32)],
        compiler_params=cparams,
    )(x2, w1m)

    conv2, st2 = pl.pallas_call(
        functools.partial(_conv2_kernel, w=w, m=m),
        grid=(n,),
        in_specs=[pl.BlockSpec((1, hw, cout), lambda i: (i, 0, 0)),
                  pl.BlockSpec((n, 8, cout), lambda i: (0, 0, 0)),
                  pl.BlockSpec((1, cout), lambda i: (0, 0)),
                  pl.BlockSpec((1, cout), lambda i: (0, 0)),
                  pl.BlockSpec((9 * cout, cout), lambda i: (0, 0))],
        out_specs=[pl.BlockSpec((1, hw, cout), lambda i: (i, 0, 0)),
                   pl.BlockSpec((1, 8, cout), lambda i: (i, 0, 0))],
        out_shape=[jax.ShapeDtypeStruct((n, hw, cout), jnp.bfloat16),
                   jax.ShapeDtypeStruct((n, 8, cout), jnp.float32)],
        compiler_params=cparams,
    )(conv1, st1, g1c, be1c, w2m)

    out = pl.pallas_call(
        functools.partial(_pool_kernel, h=h, w=w, m=m),
        grid=(n,),
        in_specs=[pl.BlockSpec((1, hw, cout), lambda i: (i, 0, 0)),
                  pl.BlockSpec((n, 8, cout), lambda i: (0, 0, 0)),
                  pl.BlockSpec((1, cout), lambda i: (0, 0)),
                  pl.BlockSpec((1, cout), lambda i: (0, 0))],
        out_specs=pl.BlockSpec((1, hw // 4, cout), lambda i: (i, 0, 0)),
        out_shape=jax.ShapeDtypeStruct((n, hw // 4, cout), jnp.float32),
        compiler_params=cparams,
    )(conv2, st2, g2c, be2c)

    return jnp.transpose(out, (0, 2, 1)).reshape(n, cout, h // 2, w // 2)


# speedup vs baseline: 2.9252x; 2.1027x over previous
import functools

import jax
import jax.numpy as jnp
import numpy as np
from jax import lax
from jax.experimental import pallas as pl
from jax.experimental.pallas import tpu as pltpu

_BN_EPS = 1e-5
_INV_SQRT2 = np.float32(1.0 / np.sqrt(2.0))
_MARGIN = 80


def _gelu(y):
    return 0.5 * y * (1.0 + lax.erf(y * _INV_SQRT2))


def _premask(src, w, row_phase):
    rows, c = src.shape
    ri = (lax.broadcasted_iota(jnp.int32, (rows, c), 0) + row_phase) % w
    zero = jnp.zeros((), src.dtype)
    src_l = jnp.where(ri != (w - 1), src, zero)
    src_r = jnp.where(ri != 0, src, zero)
    return (src_l, src, src_r)


def _patches(srcs, w, out_rows):
    taps = []
    for dy in range(3):
        for dx in range(3):
            o = (dy - 1) * w + (dx - 1)
            taps.append(srcs[dx][_MARGIN + o:_MARGIN + o + out_rows, :])
    return jnp.concatenate(taps, axis=1)


def _stats_rows(acc):
    s = jnp.sum(acc, axis=0, keepdims=True)
    ss = jnp.sum(acc * acc, axis=0, keepdims=True)
    return jnp.concatenate(
        [s, ss, jnp.zeros((6, acc.shape[1]), jnp.float32)], axis=0)


def _scale_shift(st_all, g, b, m):
    st = jnp.sum(st_all, axis=0)
    mean = st[0:1, :] / m
    var = jnp.maximum(st[1:2, :] / m - mean * mean, 0.0)
    inv = lax.rsqrt(var + _BN_EPS)
    scale = g * inv
    shift = b - mean * scale
    return scale, shift


def _fused_kernel(x_ref, w1_ref, w2_ref, g1_ref, b1_ref, o_ref, s2_ref,
                  scr_ref, st1_ref, *, w, m):
    ph = pl.program_id(0)
    j = pl.program_id(1)
    hw, cin = x_ref.shape[1], x_ref.shape[2]
    cout = w1_ref.shape[1]
    half = hw // 2
    ext = half + 2 * _MARGIN
    img = j // 2
    h2 = j % 2
    base = h2 * half

    @pl.when(ph == 0)
    def _conv1_phase():
        xb = x_ref[0].astype(jnp.bfloat16)
        zm = jnp.zeros((_MARGIN, cin), jnp.bfloat16)
        xp = jnp.concatenate([zm, xb, zm], axis=0)
        xs = jnp.where(h2 == 0, xp[:ext, :], xp[half:, :])
        p = _patches(_premask(xs, w, (-_MARGIN) % w), w, half)
        acc = jnp.dot(p, w1_ref[...], preferred_element_type=jnp.float32)
        scr_ref[img, pl.ds(pl.multiple_of(_MARGIN + base, 16), half), :] = acc.astype(jnp.bfloat16)
        moff = jnp.where(h2 == 0, 0, hw + _MARGIN)
        scr_ref[img, pl.ds(pl.multiple_of(moff, 16), _MARGIN), :] = jnp.zeros(
            (_MARGIN, cout), jnp.bfloat16)
        st1_ref[j] = _stats_rows(acc)

    @pl.when(ph == 1)
    def _conv2_phase():
        scale, shift = _scale_shift(st1_ref[...], g1_ref[...], b1_ref[...], m)
        ys = scr_ref[img, pl.ds(pl.multiple_of(base, 16), ext), :]
        y = _gelu(ys.astype(jnp.float32) * scale + shift)
        g_row = lax.broadcasted_iota(jnp.int32, (ext, cout), 0) + base
        valid = (g_row >= _MARGIN) & (g_row < hw + _MARGIN)
        yb = jnp.where(valid, y, 0.0).astype(jnp.bfloat16)
        p = _patches(_premask(yb, w, (-_MARGIN) % w), w, half)
        acc = jnp.dot(p, w2_ref[...], preferred_element_type=jnp.float32)
        o_ref[0] = acc.astype(jnp.bfloat16)
        s2_ref[0] = _stats_rows(acc)


def _pool_kernel(h_ref, st_ref, g_ref, b_ref, o_ref, *, h, w, m):
    scale, shift = _scale_shift(st_ref[...], g_ref[...], b_ref[...], m)
    y = _gelu(h_ref[0].astype(jnp.float32) * scale + shift)
    c = y.shape[1]
    yb = y.astype(jnp.bfloat16)
    z = pltpu.bitcast(yb, jnp.int32)
    even = pltpu.bitcast(z << 16, jnp.float32)
    odd = pltpu.bitcast(z & jnp.int32(-65536), jnp.float32)
    wm = jnp.maximum(even, odd)
    wm = wm.reshape(h // 2, 2, (w // 2), c)
    hm = jnp.maximum(wm[:, 0], wm[:, 1])
    o_ref[0] = hm.reshape((h // 2) * (w // 2), c)


def kernel(x, w1, b1, g1, be1, w2, b2, g2, be2):
    n, cin, h, w = x.shape
    cout = g1.shape[0]
    hw = h * w
    half = hw // 2
    m = float(n * hw)

    x2 = jnp.transpose(x.reshape(n, cin, hw), (0, 2, 1))
    w1m = jnp.transpose(w1, (2, 3, 1, 0)).reshape(9 * cin, cout).astype(jnp.bfloat16)
    w2m = jnp.transpose(w2, (2, 3, 1, 0)).reshape(9 * cout, cout).astype(jnp.bfloat16)
    g1c, be1c = g1.reshape(1, cout), be1.reshape(1, cout)
    g2c, be2c = g2.reshape(1, cout), be2.reshape(1, cout)

    cparams = pltpu.CompilerParams(
        dimension_semantics=("arbitrary", "arbitrary"),
        vmem_limit_bytes=63 * 1024 * 1024,
    )

    conv2, st2 = pl.pallas_call(
        functools.partial(_fused_kernel, w=w, m=m),
        grid=(2, 2 * n),
        in_specs=[
            pl.BlockSpec((1, hw, cin),
                         lambda p, j: (jnp.where(p == 0, j // 2, 0), 0, 0)),
            pl.BlockSpec((9 * cin, cout), lambda p, j: (0, 0)),
            pl.BlockSpec((9 * cout, cout), lambda p, j: (0, 0)),
            pl.BlockSpec((1, cout), lambda p, j: (0, 0)),
            pl.BlockSpec((1, cout), lambda p, j: (0, 0)),
        ],
        out_specs=[
            pl.BlockSpec((1, half, cout),
                         lambda p, j: (jnp.where(p == 1, j, 0), 0, 0)),
            pl.BlockSpec((1, 8, cout),
                         lambda p, j: (jnp.where(p == 1, j, 0), 0, 0)),
        ],
        out_shape=[jax.ShapeDtypeStruct((2 * n, half, cout), jnp.bfloat16),
                   jax.ShapeDtypeStruct((2 * n, 8, cout), jnp.float32)],
        scratch_shapes=[
            pltpu.VMEM((n, hw + 2 * _MARGIN, cout), jnp.bfloat16),
            pltpu.VMEM((2 * n, 8, cout), jnp.float32),
        ],
        compiler_params=cparams,
    )(x2, w1m, w2m, g1c, be1c)

    out = pl.pallas_call(
        functools.partial(_pool_kernel, h=h, w=w, m=m),
        grid=(n,),
        in_specs=[pl.BlockSpec((1, hw, cout), lambda i: (i, 0, 0)),
                  pl.BlockSpec((2 * n, 8, cout), lambda i: (0, 0, 0)),
                  pl.BlockSpec((1, cout), lambda i: (0, 0)),
                  pl.BlockSpec((1, cout), lambda i: (0, 0))],
        out_specs=pl.BlockSpec((1, hw // 4, cout), lambda i: (i, 0, 0)),
        out_shape=jax.ShapeDtypeStruct((n, hw // 4, cout), jnp.float32),
        compiler_params=pltpu.CompilerParams(
            dimension_semantics=("arbitrary",),
            vmem_limit_bytes=63 * 1024 * 1024,
        ),
    )(conv2.reshape(n, hw, cout), st2, g2c, be2c)

    return jnp.transpose(out.reshape(n, h // 2, w // 2, cout), (0, 3, 1, 2))


# speedup vs baseline: 3.1778x; 1.0864x over previous
import functools

import jax
import jax.numpy as jnp
import numpy as np
from jax import lax
from jax.experimental import pallas as pl
from jax.experimental.pallas import tpu as pltpu

_BN_EPS = 1e-5
_INV_SQRT2 = np.float32(1.0 / np.sqrt(2.0))
_PAD = 128


def _gelu(y):
    return 0.5 * y * (1.0 + lax.erf(y * _INV_SQRT2))


def _taps_rmajor(yb, w):
    hw, c = yb.shape
    zpad = jnp.zeros((_PAD, c), yb.dtype)
    yp = jnp.concatenate([zpad, yb, zpad], axis=0)
    ri = lax.broadcasted_iota(jnp.int32, (hw + 2 * _PAD, c), 0) % w
    zero = jnp.zeros((), yb.dtype)
    yp_l = jnp.where(ri != (w - 1), yp, zero)
    yp_r = jnp.where(ri != 0, yp, zero)
    srcs = (yp_l, yp, yp_r)
    taps = []
    for dy in range(3):
        for dx in range(3):
            o = (dy - 1) * w + (dx - 1)
            taps.append(srcs[dx][_PAD + o:_PAD + o + hw, :])
    return taps


def _stats_rows(acc):
    s = jnp.sum(acc, axis=0, keepdims=True)
    ss = jnp.sum(acc * acc, axis=0, keepdims=True)
    return jnp.concatenate(
        [s, ss, jnp.zeros((6, acc.shape[1]), jnp.float32)], axis=0)


def _scale_shift(st_all, g, b, m):
    st = jnp.sum(st_all, axis=0)
    mean = st[0:1, :] / m
    var = jnp.maximum(st[1:2, :] / m - mean * mean, 0.0)
    inv = lax.rsqrt(var + _BN_EPS)
    scale = g * inv
    shift = b - mean * scale
    return scale, shift


def _conv1_kernel(x_ref, w_ref, o_ref, s_ref, *, w):
    xt = x_ref[0].astype(jnp.bfloat16)
    p = jnp.concatenate(_taps_rmajor(xt, w), axis=1)
    acc = jnp.dot(p, w_ref[...], preferred_element_type=jnp.float32)
    o_ref[0] = acc.astype(jnp.bfloat16)
    s_ref[0] = _stats_rows(acc)


def _conv2_kernel(h_ref, st_ref, g_ref, b_ref, w_ref, o_ref, s_ref, *, w, m):
    scale, shift = _scale_shift(st_ref[...], g_ref[...], b_ref[...], m)
    y = h_ref[0].astype(jnp.float32) * scale + shift
    yb = _gelu(y).astype(jnp.bfloat16)
    taps = _taps_rmajor(yb, w)
    cin = yb.shape[1]
    acc = None
    for dy in range(3):
        p = jnp.concatenate(taps[3 * dy:3 * dy + 3], axis=1)
        d = jnp.dot(p, w_ref[3 * dy * cin:(3 * dy + 3) * cin, :],
                    preferred_element_type=jnp.float32)
        acc = d if acc is None else acc + d
    o_ref[0] = acc.astype(jnp.bfloat16)
    s_ref[0] = _stats_rows(acc)


def _pool_kernel(h_ref, st_ref, g_ref, b_ref, o_ref, *, h, w, m):
    scale, shift = _scale_shift(st_ref[...], g_ref[...], b_ref[...], m)
    y = _gelu(h_ref[0].astype(jnp.float32) * scale + shift)
    c = y.shape[1]
    yb = y.astype(jnp.bfloat16)
    z = pltpu.bitcast(yb, jnp.int32)
    even = pltpu.bitcast(z << 16, jnp.float32)
    odd = pltpu.bitcast(z & jnp.int32(-65536), jnp.float32)
    wm = jnp.maximum(even, odd)
    wm = wm.reshape(h // 2, 2, (w // 2), c)
    hm = jnp.maximum(wm[:, 0], wm[:, 1])
    o_ref[0] = hm.reshape((h // 2) * (w // 2), c)


def kernel(x, w1, b1, g1, be1, w2, b2, g2, be2):
    n, cin, h, w = x.shape
    cout = g1.shape[0]
    hw = h * w
    m = float(n * hw)

    x2 = jnp.transpose(x.reshape(n, cin, hw), (0, 2, 1))
    w1m = jnp.transpose(w1, (2, 3, 1, 0)).reshape(9 * cin, cout).astype(jnp.bfloat16)
    w2m = jnp.transpose(w2, (2, 3, 1, 0)).reshape(9 * cout, cout).astype(jnp.bfloat16)
    g1c, be1c = g1.reshape(1, cout), be1.reshape(1, cout)
    g2c, be2c = g2.reshape(1, cout), be2.reshape(1, cout)

    cparams = pltpu.CompilerParams(
        dimension_semantics=("parallel",),
        vmem_limit_bytes=100 * 1024 * 1024,
    )

    conv1, st1 = pl.pallas_call(
        functools.partial(_conv1_kernel, w=w),
        grid=(n,),
        in_specs=[pl.BlockSpec((1, hw, cin), lambda i: (i, 0, 0)),
                  pl.BlockSpec((9 * cin, cout), lambda i: (0, 0))],
        out_specs=[pl.BlockSpec((1, hw, cout), lambda i: (i, 0, 0)),
                   pl.BlockSpec((1, 8, cout), lambda i: (i, 0, 0))],
        out_shape=[jax.ShapeDtypeStruct((n, hw, cout), jnp.bfloat16),
                   jax.ShapeDtypeStruct((n, 8, cout), jnp.float32)],
        compiler_params=cparams,
    )(x2, w1m)

    conv2, st2 = pl.pallas_call(
        functools.partial(_conv2_kernel, w=w, m=m),
        grid=(n,),
        in_specs=[pl.BlockSpec((1, hw, cout), lambda i: (i, 0, 0)),
                  pl.BlockSpec((n, 8, cout), lambda i: (0, 0, 0)),
                  pl.BlockSpec((1, cout), lambda i: (0, 0)),
                  pl.BlockSpec((1, cout), lambda i: (0, 0)),
                  pl.BlockSpec((9 * cout, cout), lambda i: (0, 0))],
        out_specs=[pl.BlockSpec((1, hw, cout), lambda i: (i, 0, 0)),
                   pl.BlockSpec((1, 8, cout), lambda i: (i, 0, 0))],
        out_shape=[jax.ShapeDtypeStruct((n, hw, cout), jnp.bfloat16),
                   jax.ShapeDtypeStruct((n, 8, cout), jnp.float32)],
        compiler_params=cparams,
    )(conv1, st1, g1c, be1c, w2m)

    out = pl.pallas_call(
        functools.partial(_pool_kernel, h=h, w=w, m=m),
        grid=(n,),
        in_specs=[pl.BlockSpec((1, hw, cout), lambda i: (i, 0, 0)),
                  pl.BlockSpec((n, 8, cout), lambda i: (0, 0, 0)),
                  pl.BlockSpec((1, cout), lambda i: (0, 0)),
                  pl.BlockSpec((1, cout), lambda i: (0, 0))],
        out_specs=pl.BlockSpec((1, hw // 4, cout), lambda i: (i, 0, 0)),
        out_shape=jax.ShapeDtypeStruct((n, hw // 4, cout), jnp.float32),
        compiler_params=cparams,
    )(conv2, st2, g2c, be2c)

    return jnp.transpose(out.reshape(n, h // 2, w // 2, cout), (0, 3, 1, 2))
